# initial kernel scaffold (unmeasured)
import jax
import jax.numpy as jnp
from jax import lax
from jax.experimental import pallas as pl
from jax.experimental.pallas import tpu as pltpu

N_DEV = 4
M_PER = 2048
K_PER = 2048
K = 8192
N = 4096


def _a2a_body(x_ref, out_ref, copy_sem, send_sems, recv_sems):
    p = lax.axis_index("i")

    barrier_sem = pltpu.get_barrier_semaphore()
    for d in range(1, N_DEV):
        peer = lax.rem(p + d, N_DEV)
        pl.semaphore_signal(
            barrier_sem,
            inc=1,
            device_id=(peer,),
            device_id_type=pl.DeviceIdType.MESH,
        )
    pl.semaphore_wait(barrier_sem, N_DEV - 1)

    local = pltpu.make_async_copy(
        x_ref.at[pl.ds(p * M_PER, M_PER), :],
        out_ref.at[:, pl.ds(p * K_PER, K_PER)],
        copy_sem,
    )
    local.start()

    rdmas = []
    for d in range(1, N_DEV):
        q = lax.rem(p + d, N_DEV)
        rdma = pltpu.make_async_remote_copy(
            src_ref=x_ref.at[pl.ds(q * M_PER, M_PER), :],
            dst_ref=out_ref.at[:, pl.ds(p * K_PER, K_PER)],
            send_sem=send_sems.at[d - 1],
            recv_sem=recv_sems.at[d - 1],
            device_id=(q,),
            device_id_type=pl.DeviceIdType.MESH,
        )
        rdma.start()
        rdmas.append(rdma)

    local.wait()
    for rdma in rdmas:
        rdma.wait()


def _a2a(x_shard):
    return pl.pallas_call(
        _a2a_body,
        out_shape=jax.ShapeDtypeStruct((M_PER, K), jnp.float32),
        in_specs=[pl.BlockSpec(memory_space=pltpu.ANY)],
        out_specs=pl.BlockSpec(memory_space=pltpu.ANY),
        scratch_shapes=[
            pltpu.SemaphoreType.DMA,
            pltpu.SemaphoreType.DMA((N_DEV - 1,)),
            pltpu.SemaphoreType.DMA((N_DEV - 1,)),
        ],
        compiler_params=pltpu.CompilerParams(collective_id=0),
    )(x_shard)


def _gelu(y):
    c = 0.7978845608028654
    return (0.5 * y * (1.0 + jnp.tanh(c * (y + 0.044715 * y**3)))).astype(
        jnp.float32
    )


def kernel(x, w_mat):
    x_full = _a2a(x)
    y = jnp.dot(x_full, w_mat, preferred_element_type=jnp.float32)
    return _gelu(y)


# baseline (device time: 710297 ns/iter reference)
import jax
import jax.numpy as jnp
from jax import lax
from jax.experimental import pallas as pl
from jax.experimental.pallas import tpu as pltpu

N_DEV = 4
M_PER = 2048
K_PER = 2048
K = 8192
N = 4096


def _a2a_body(x_ref, out_ref, copy_sem, send_sems, recv_sems):
    p = lax.axis_index("i")

    barrier_sem = pltpu.get_barrier_semaphore()
    for d in range(1, N_DEV):
        peer = lax.rem(p + d, N_DEV)
        pl.semaphore_signal(
            barrier_sem,
            inc=1,
            device_id=(peer,),
            device_id_type=pl.DeviceIdType.MESH,
        )
    pl.semaphore_wait(barrier_sem, N_DEV - 1)

    local = pltpu.make_async_copy(
        x_ref.at[pl.ds(p * M_PER, M_PER), :],
        out_ref.at[:, pl.ds(p * K_PER, K_PER)],
        copy_sem,
    )
    local.start()

    rdmas = []
    for d in range(1, N_DEV):
        q = lax.rem(p + d, N_DEV)
        rdma = pltpu.make_async_remote_copy(
            src_ref=x_ref.at[pl.ds(q * M_PER, M_PER), :],
            dst_ref=out_ref.at[:, pl.ds(p * K_PER, K_PER)],
            send_sem=send_sems.at[d - 1],
            recv_sem=recv_sems.at[d - 1],
            device_id=(q,),
            device_id_type=pl.DeviceIdType.MESH,
        )
        rdma.start()
        rdmas.append(rdma)

    local.wait()
    for rdma in rdmas:
        rdma.wait()


def _a2a(x_shard):
    return pl.pallas_call(
        _a2a_body,
        out_shape=jax.ShapeDtypeStruct((M_PER, K), jnp.float32),
        in_specs=[pl.BlockSpec(memory_space=pl.ANY)],
        out_specs=pl.BlockSpec(memory_space=pl.ANY),
        scratch_shapes=[
            pltpu.SemaphoreType.DMA,
            pltpu.SemaphoreType.DMA((N_DEV - 1,)),
            pltpu.SemaphoreType.DMA((N_DEV - 1,)),
        ],
        compiler_params=pltpu.CompilerParams(collective_id=0),
    )(x_shard)


def _gelu(y):
    c = 0.7978845608028654
    return (0.5 * y * (1.0 + jnp.tanh(c * (y + 0.044715 * y**3)))).astype(
        jnp.float32
    )


def kernel(x, w_mat):
    x_full = _a2a(x)
    y = jnp.dot(x_full, w_mat, preferred_element_type=jnp.float32)
    return _gelu(y)


# device time: 476846 ns/iter; 1.4896x vs baseline; 1.4896x over previous
import jax
import jax.numpy as jnp
from jax import lax
from jax.experimental import pallas as pl
from jax.experimental.pallas import tpu as pltpu

N_DEV = 4
M_PER = 2048
K_PER = 2048
K = 8192
N = 4096


def _a2a_body(x_ref, out_ref, copy_sem, send_sems, recv_sems):
    p = lax.axis_index("i")

    barrier_sem = pltpu.get_barrier_semaphore()
    for d in range(1, N_DEV):
        peer = lax.rem(p + d, N_DEV)
        pl.semaphore_signal(
            barrier_sem,
            inc=1,
            device_id=(peer,),
            device_id_type=pl.DeviceIdType.MESH,
        )
    pl.semaphore_wait(barrier_sem, N_DEV - 1)

    local = pltpu.make_async_copy(
        x_ref.at[pl.ds(p * M_PER, M_PER), :],
        out_ref.at[:, pl.ds(p * K_PER, K_PER)],
        copy_sem,
    )
    local.start()

    rdmas = []
    for d in range(1, N_DEV):
        q = lax.rem(p + d, N_DEV)
        rdma = pltpu.make_async_remote_copy(
            src_ref=x_ref.at[pl.ds(q * M_PER, M_PER), :],
            dst_ref=out_ref.at[:, pl.ds(p * K_PER, K_PER)],
            send_sem=send_sems.at[d - 1],
            recv_sem=recv_sems.at[d - 1],
            device_id=(q,),
            device_id_type=pl.DeviceIdType.MESH,
        )
        rdma.start()
        rdmas.append(rdma)

    local.wait()
    for rdma in rdmas:
        rdma.wait()


def _a2a(x_shard):
    return pl.pallas_call(
        _a2a_body,
        out_shape=jax.ShapeDtypeStruct((M_PER, K), x_shard.dtype),
        in_specs=[pl.BlockSpec(memory_space=pl.ANY)],
        out_specs=pl.BlockSpec(memory_space=pl.ANY),
        scratch_shapes=[
            pltpu.SemaphoreType.DMA,
            pltpu.SemaphoreType.DMA((N_DEV - 1,)),
            pltpu.SemaphoreType.DMA((N_DEV - 1,)),
        ],
        compiler_params=pltpu.CompilerParams(collective_id=0),
    )(x_shard)


def _gelu(y):
    c = 0.7978845608028654
    return (0.5 * y * (1.0 + jnp.tanh(c * (y + 0.044715 * y**3)))).astype(
        jnp.float32
    )


def kernel(x, w_mat):
    x_full = _a2a(x.astype(jnp.bfloat16))
    y = jnp.dot(
        x_full, w_mat.astype(jnp.bfloat16), preferred_element_type=jnp.float32
    )
    return _gelu(y)


# device time: 363005 ns/iter; 1.9567x vs baseline; 1.3136x over previous
import jax
import jax.numpy as jnp
from jax import lax
from jax.experimental import pallas as pl
from jax.experimental.pallas import tpu as pltpu

jax.config.update("jax_compilation_cache_dir", "/tmp/scband_jax_cache")
jax.config.update("jax_persistent_cache_min_compile_time_secs", 5.0)

N_DEV = 4
M_PER = 2048
K_PER = 2048
K = 8192
N = 4096
MH = M_PER // 2
NT = 1024
N_NT = N // NT


def _gelu(y):
    c = 0.7978845608028654
    return 0.5 * y * (1.0 + jnp.tanh(c * (y + 0.044715 * y**3)))


def _fused_body(
    x_ref,
    w_ref,
    out_ref,
    recv_ref,
    acc_ref,
    wbuf_ref,
    send_sems,
    recv_sems,
    x_sem,
    w_sems,
    out_sem,
):
    p = lax.axis_index("i")

    barrier_sem = pltpu.get_barrier_semaphore()
    for d in range(1, N_DEV):
        peer = lax.rem(p + d, N_DEV)
        pl.semaphore_signal(
            barrier_sem,
            inc=1,
            device_id=(peer,),
            device_id_type=pl.DeviceIdType.MESH,
        )
    pl.semaphore_wait(barrier_sem, N_DEV - 1)

    local_cp = pltpu.make_async_copy(
        x_ref.at[pl.ds(p * M_PER, M_PER), :], recv_ref.at[0], x_sem
    )
    local_cp.start()

    rdmas = []
    for mh in range(2):
        for d in range(1, N_DEV):
            q = lax.rem(p + d, N_DEV)
            rdma = pltpu.make_async_remote_copy(
                src_ref=x_ref.at[pl.ds(q * M_PER + mh * MH, MH), :],
                dst_ref=recv_ref.at[d, pl.ds(mh * MH, MH), :],
                send_sem=send_sems.at[d - 1, mh],
                recv_sem=recv_sems.at[d - 1, mh],
                device_id=(q,),
                device_id_type=pl.DeviceIdType.MESH,
            )
            rdma.start()
            rdmas.append(rdma)

    ks = [p] + [lax.rem(p + (N_DEV - d), N_DEV) for d in (1, 2, 3)]
    step_sem = [None, 0, 1, 2]

    w_copies = []
    for mh in range(2):
        for s in range(4):
            for n in range(N_NT):
                i = len(w_copies)
                w_copies.append(
                    pltpu.make_async_copy(
                        w_ref.at[pl.ds(ks[s] * K_PER, K_PER), pl.ds(n * NT, NT)],
                        wbuf_ref.at[i % 2],
                        w_sems.at[i % 2],
                    )
                )
    w_copies[0].start()
    w_copies[1].start()

    out_cps = []
    i = 0
    for mh in range(2):
        if mh == 1:
            out_cps[0].wait()
        for s in range(4):
            if step_sem[s] is None:
                if mh == 0:
                    local_cp.wait()
            else:
                pltpu.make_async_remote_copy(
                    src_ref=x_ref.at[pl.ds(0, MH), :],
                    dst_ref=recv_ref.at[s, pl.ds(mh * MH, MH), :],
                    send_sem=send_sems.at[step_sem[s], mh],
                    recv_sem=recv_sems.at[step_sem[s], mh],
                    device_id=(p,),
                    device_id_type=pl.DeviceIdType.MESH,
                ).wait_recv()
            for n in range(N_NT):
                w_copies[i].wait()
                lhs = recv_ref[s, mh * MH : (mh + 1) * MH, :]
                rhs = wbuf_ref[i % 2]
                partial = jnp.dot(lhs, rhs, preferred_element_type=jnp.float32)
                nsl = slice(n * NT, (n + 1) * NT)
                if s == 0:
                    acc_ref[:, nsl] = partial
                else:
                    acc_ref[:, nsl] += partial
                if i + 2 < len(w_copies):
                    w_copies[i + 2].start()
                i += 1
        for n in range(N_NT):
            nsl = slice(n * NT, (n + 1) * NT)
            acc_ref[:, nsl] = _gelu(acc_ref[:, nsl])
        out_cp = pltpu.make_async_copy(
            acc_ref, out_ref.at[pl.ds(mh * MH, MH), :], out_sem
        )
        out_cp.start()
        out_cps.append(out_cp)

    out_cps[1].wait()
    for rdma in rdmas:
        rdma.wait_send()


def _fused(xb, wb):
    return pl.pallas_call(
        _fused_body,
        out_shape=jax.ShapeDtypeStruct((M_PER, N), jnp.float32),
        in_specs=[
            pl.BlockSpec(memory_space=pl.ANY),
            pl.BlockSpec(memory_space=pl.ANY),
        ],
        out_specs=pl.BlockSpec(memory_space=pl.ANY),
        scratch_shapes=[
            pltpu.VMEM((N_DEV, M_PER, K_PER), jnp.bfloat16),
            pltpu.VMEM((MH, N), jnp.float32),
            pltpu.VMEM((2, K_PER, NT), jnp.bfloat16),
            pltpu.SemaphoreType.DMA((N_DEV - 1, 2)),
            pltpu.SemaphoreType.DMA((N_DEV - 1, 2)),
            pltpu.SemaphoreType.DMA,
            pltpu.SemaphoreType.DMA((2,)),
            pltpu.SemaphoreType.DMA,
        ],
        compiler_params=pltpu.CompilerParams(
            collective_id=0,
            vmem_limit_bytes=65472 * 1024,
        ),
    )(xb, wb)


def kernel(x, w_mat):
    return _fused(x.astype(jnp.bfloat16), w_mat.astype(jnp.bfloat16))


# device time: 312305 ns/iter; 2.2744x vs baseline; 1.1623x over previous
import jax
import jax.numpy as jnp
from jax import lax
from jax.experimental import pallas as pl
from jax.experimental.pallas import tpu as pltpu

jax.config.update("jax_compilation_cache_dir", "/tmp/scband_jax_cache")
jax.config.update("jax_persistent_cache_min_compile_time_secs", 5.0)

N_DEV = 4
M_PER = 2048
K_PER = 2048
K = 8192
N = 4096
MH = M_PER // 2
NT = 512
N_NT = N // NT


def _gelu(y):
    c = 0.7978845608028654
    return 0.5 * y * (1.0 + jnp.tanh(c * (y + 0.044715 * y**3)))


def _fused_body(
    x_ref,
    xb_ref,
    w_ref,
    out_ref,
    recv_ref,
    acc_ref,
    lhs_ref,
    wbuf_ref,
    send_sems,
    recv_sems,
    x_sem,
    w_sems,
    out_sem,
):
    p = lax.axis_index("i")

    barrier_sem = pltpu.get_barrier_semaphore()
    for d in range(1, N_DEV):
        peer = lax.rem(p + d, N_DEV)
        pl.semaphore_signal(
            barrier_sem,
            inc=1,
            device_id=(peer,),
            device_id_type=pl.DeviceIdType.MESH,
        )
    pl.semaphore_wait(barrier_sem, N_DEV - 1)

    rdmas = []
    for mh in range(2):
        for d in range(1, N_DEV):
            q = lax.rem(p + d, N_DEV)
            rdma = pltpu.make_async_remote_copy(
                src_ref=xb_ref.at[pl.ds(q * M_PER + mh * MH, MH), :],
                dst_ref=recv_ref.at[d - 1, pl.ds(mh * MH, MH), :],
                send_sem=send_sems.at[d - 1, mh],
                recv_sem=recv_sems.at[d - 1, mh],
                device_id=(q,),
                device_id_type=pl.DeviceIdType.MESH,
            )
            rdma.start()
            rdmas.append(rdma)

    ks = [p] + [lax.rem(p + (N_DEV - d), N_DEV) for d in (1, 2, 3)]

    w_copies = []
    for mh in range(2):
        for s in range(4):
            for n in range(N_NT):
                i = len(w_copies)
                w_copies.append(
                    pltpu.make_async_copy(
                        w_ref.at[pl.ds(ks[s] * K_PER, K_PER), pl.ds(n * NT, NT)],
                        wbuf_ref.at[i % 2],
                        w_sems.at[i % 2],
                    )
                )
    w_copies[0].start()
    w_copies[1].start()

    out_cps = []
    i = 0
    for mh in range(2):
        if mh == 1:
            out_cps[0].wait()
        for s in range(4):
            if s == 0:
                local_cp = pltpu.make_async_copy(
                    x_ref.at[pl.ds(p * M_PER + mh * MH, MH), :],
                    lhs_ref,
                    x_sem,
                )
                local_cp.start()
                local_cp.wait()
            else:
                pltpu.make_async_remote_copy(
                    src_ref=xb_ref.at[pl.ds(0, MH), :],
                    dst_ref=recv_ref.at[s - 1, pl.ds(mh * MH, MH), :],
                    send_sem=send_sems.at[s - 1, mh],
                    recv_sem=recv_sems.at[s - 1, mh],
                    device_id=(p,),
                    device_id_type=pl.DeviceIdType.MESH,
                ).wait_recv()
                lhs_ref[...] = recv_ref[
                    s - 1, mh * MH : (mh + 1) * MH, :
                ].astype(jnp.float32)
            for n in range(N_NT):
                w_copies[i].wait()
                partial = jnp.dot(
                    lhs_ref[...],
                    wbuf_ref[i % 2],
                    preferred_element_type=jnp.float32,
                )
                nsl = slice(n * NT, (n + 1) * NT)
                if s == 0:
                    acc_ref[:, nsl] = partial
                else:
                    acc_ref[:, nsl] += partial
                if i + 2 < len(w_copies):
                    w_copies[i + 2].start()
                i += 1
        for n in range(N_NT):
            nsl = slice(n * NT, (n + 1) * NT)
            acc_ref[:, nsl] = _gelu(acc_ref[:, nsl])
        out_cp = pltpu.make_async_copy(
            acc_ref, out_ref.at[pl.ds(mh * MH, MH), :], out_sem
        )
        out_cp.start()
        out_cps.append(out_cp)

    out_cps[1].wait()
    for rdma in rdmas:
        rdma.wait_send()


def _fused(x, xb, w):
    return pl.pallas_call(
        _fused_body,
        out_shape=jax.ShapeDtypeStruct((M_PER, N), jnp.float32),
        in_specs=[
            pl.BlockSpec(memory_space=pl.ANY),
            pl.BlockSpec(memory_space=pl.ANY),
            pl.BlockSpec(memory_space=pl.ANY),
        ],
        out_specs=pl.BlockSpec(memory_space=pl.ANY),
        scratch_shapes=[
            pltpu.VMEM((N_DEV - 1, M_PER, K_PER), jnp.bfloat16),
            pltpu.VMEM((MH, N), jnp.float32),
            pltpu.VMEM((MH, K_PER), jnp.float32),
            pltpu.VMEM((2, K_PER, NT), jnp.float32),
            pltpu.SemaphoreType.DMA((N_DEV - 1, 2)),
            pltpu.SemaphoreType.DMA((N_DEV - 1, 2)),
            pltpu.SemaphoreType.DMA,
            pltpu.SemaphoreType.DMA((2,)),
            pltpu.SemaphoreType.DMA,
        ],
        compiler_params=pltpu.CompilerParams(
            collective_id=0,
            vmem_limit_bytes=65472 * 1024,
        ),
    )(x, xb, w)


def kernel(x, w_mat):
    return _fused(x, x.astype(jnp.bfloat16), w_mat)
